# baseline (speedup 1.0000x reference)
import jax
import jax.numpy as jnp
from jax.experimental import pallas as pl


def _lobula_kernel(tm3_ref, tm1p3_ref, tm2_ref, mi1p3_ref, on_ref, off_ref):
    on_ref[...] = tm3_ref[...] * tm1p3_ref[...]
    off_ref[...] = tm2_ref[...] * mi1p3_ref[...]


def kernel(tm3Signal, tm2Signal, Mi1Para5Signal, tm1Para5Signal, tm1Para3Signal, Mi1Para3Signal):
    H, W = tm3Signal.shape[2], tm3Signal.shape[3]
    shape2d = (H, W)
    a = tm3Signal.reshape(shape2d)
    b = tm1Para3Signal.reshape(shape2d)
    c = tm2Signal.reshape(shape2d)
    d = Mi1Para3Signal.reshape(shape2d)
    out_sd = jax.ShapeDtypeStruct(shape2d, tm3Signal.dtype)
    n_tiles = 8
    rows = H // n_tiles
    spec = pl.BlockSpec((rows, W), lambda i: (i, 0))
    on2d, off2d = pl.pallas_call(
        _lobula_kernel,
        grid=(n_tiles,),
        in_specs=[spec, spec, spec, spec],
        out_specs=(spec, spec),
        out_shape=(out_sd, out_sd),
    )(a, b, c, d)
    return (on2d.reshape(1, 1, H, W), off2d.reshape(1, 1, H, W))


# speedup vs baseline: 1.8016x; 1.8016x over previous
import jax
import jax.numpy as jnp
from jax.experimental import pallas as pl


def _lobula_kernel(tm3_ref, tm1p3_ref, tm2_ref, mi1p3_ref, on_ref, off_ref):
    on_ref[...] = tm3_ref[...] * tm1p3_ref[...]
    off_ref[...] = tm2_ref[...] * mi1p3_ref[...]


def kernel(tm3Signal, tm2Signal, Mi1Para5Signal, tm1Para5Signal, tm1Para3Signal, Mi1Para3Signal):
    H, W = tm3Signal.shape[2], tm3Signal.shape[3]
    shape2d = (H, W)
    a = tm3Signal.reshape(shape2d)
    b = tm1Para3Signal.reshape(shape2d)
    c = tm2Signal.reshape(shape2d)
    d = Mi1Para3Signal.reshape(shape2d)
    out_sd = jax.ShapeDtypeStruct(shape2d, tm3Signal.dtype)
    n_tiles = 2
    rows = H // n_tiles
    spec = pl.BlockSpec((rows, W), lambda i: (i, 0))
    on2d, off2d = pl.pallas_call(
        _lobula_kernel,
        grid=(n_tiles,),
        in_specs=[spec, spec, spec, spec],
        out_specs=(spec, spec),
        out_shape=(out_sd, out_sd),
    )(a, b, c, d)
    return (on2d.reshape(1, 1, H, W), off2d.reshape(1, 1, H, W))
